# baseline (speedup 1.0000x reference)
import functools

import numpy as np
import jax
import jax.numpy as jnp
from jax import lax
from jax.experimental import pallas as pl
from jax.experimental.pallas import tpu as pltpu
from jax.experimental.pallas import tpu_sc as plsc

B = 4096
INPUT_ROWS = 260001
F = 26
D = 64
E_COLS = F * D
NPAIR = F * (F - 1) // 2
PAIR_PAD = 384
H1 = 400
H2 = 400
BN_EPS = 1e-3

_OFFSETS = np.arange(F, dtype=np.int32) * 10000

_pairs = [(i, i + o) for o in range(1, F) for i in range(F - o)]
PERM = np.array([i * 25 - i * (i - 1) // 2 + (j - i - 1) for i, j in _pairs],
                dtype=np.int32)

NW = 32
ROWS = B * F
RPW = ROWS // NW
CHUNK = 128
KFIRE = 13
HALF = KFIRE * CHUNK


@functools.lru_cache(maxsize=None)
def _make_gather_sc():
    @functools.partial(
        pl.kernel,
        mesh=plsc.VectorSubcoreMesh(core_axis_name="c", subcore_axis_name="s"),
        out_type=jax.ShapeDtypeStruct((ROWS, D), jnp.float32),
        scratch_types=[
            pltpu.VMEM((RPW,), jnp.int32),
            pltpu.VMEM((HALF, D), jnp.float32),
            pltpu.SemaphoreType.DMA,
        ],
        compiler_params=pltpu.CompilerParams(use_tc_tiling_on_sc=False),
    )
    def _gather_sc(table_hbm, idx_hbm, out_hbm, idx_v, rows_v, sem):
        wid = lax.axis_index("s") * 2 + lax.axis_index("c")
        base = wid * RPW
        pltpu.sync_copy(idx_hbm.at[pl.ds(base, RPW)], idx_v)
        for h in range(RPW // HALF):
            cps = []
            for c in range(KFIRE):
                cp = pltpu.make_async_copy(
                    table_hbm.at[idx_v.at[pl.ds(h * HALF + c * CHUNK, CHUNK)]],
                    rows_v.at[pl.ds(c * CHUNK, CHUNK), :],
                    sem,
                )
                cp.start()
                cps.append(cp)
            for cp in cps:
                cp.wait()
            pltpu.sync_copy(rows_v, out_hbm.at[pl.ds(base + h * HALF, HALF)])

    return _gather_sc


BB = 512

_seg_w = [F - o for o in range(1, F)]
_seg_w8 = [-(-w // 8) * 8 for w in _seg_w]
_SEG_OFF = np.cumsum([0] + _seg_w8[:-1]).astype(np.int32)
PADK = int(np.sum(_seg_w8))
_PAD_POS = np.concatenate(
    [_SEG_OFF[o - 1] + np.arange(F - o) for o in range(1, F)]).astype(np.int32)


def _dot00(a, b):
    return lax.dot_general(a, b, (((0,), (0,)), ((), ())),
                           preferred_element_type=jnp.float32)


def _tc_body(e2_ref, w1a_ref, w1b_ref, s1_ref, b1_ref, w2_ref, s2_ref,
             b2_ref, w3_ref, b3_ref, out_ref):
    et = jnp.transpose(e2_ref[...])
    pieces = []
    for o in range(1, F):
        w = F - o
        c = et[:w * D, :] * et[o * D:, :]
        s = jnp.sum(c.reshape(w, D, BB), axis=1)
        pad = _seg_w8[o - 1] - w
        if pad:
            s = jnp.concatenate(
                [s, jnp.zeros((pad, BB), jnp.float32)], axis=0)
        pieces.append(s)
    prod_t = jnp.concatenate(pieces, axis=0)
    h = _dot00(w1a_ref[...], et)
    h += _dot00(w1b_ref[...], prod_t)
    h = jnp.maximum(h * s1_ref[...] + b1_ref[...], 0.0)
    h = _dot00(w2_ref[...], h)
    h = jnp.maximum(h * s2_ref[...] + b2_ref[...], 0.0)
    z = _dot00(w3_ref[...], h) + b3_ref[...]
    out_ref[...] = jnp.transpose(jax.nn.sigmoid(z))


_tc_call = pl.pallas_call(
    _tc_body,
    grid=(B // BB,),
    in_specs=[
        pl.BlockSpec((BB, E_COLS), lambda i: (i, 0)),
        pl.BlockSpec((E_COLS, H1), lambda i: (0, 0)),
        pl.BlockSpec((PADK, H1), lambda i: (0, 0)),
        pl.BlockSpec((H1, 1), lambda i: (0, 0)),
        pl.BlockSpec((H1, 1), lambda i: (0, 0)),
        pl.BlockSpec((H1, H2), lambda i: (0, 0)),
        pl.BlockSpec((H2, 1), lambda i: (0, 0)),
        pl.BlockSpec((H2, 1), lambda i: (0, 0)),
        pl.BlockSpec((H2, 1), lambda i: (0, 0)),
        pl.BlockSpec((1, 1), lambda i: (0, 0)),
    ],
    out_specs=pl.BlockSpec((BB, 1), lambda i: (i, 0)),
    out_shape=jax.ShapeDtypeStruct((B, 1), jnp.float32),
)


def kernel(x, emb_table, W1, b1, g1, be1, W2, b2, g2, be2, W3, b3):
    idx = (x.astype(jnp.int32) + _OFFSETS[None, :]).reshape(-1)
    e = _make_gather_sc()(emb_table, idx)

    s1 = (g1 * (1.0 / np.sqrt(1.0 + BN_EPS))).reshape(H1, 1)
    b1f = b1.reshape(H1, 1) * s1 + be1.reshape(H1, 1)
    s2 = (g2 * (1.0 / np.sqrt(1.0 + BN_EPS))).reshape(H2, 1)
    b2f = b2.reshape(H2, 1) * s2 + be2.reshape(H2, 1)
    w1bp = jnp.zeros((PADK, H1), jnp.float32).at[_PAD_POS].set(
        W1[E_COLS:][PERM])

    out = _tc_call(e.reshape(B, E_COLS),
                   W1, w1bp, s1, b1f, W2, s2, b2f, W3, b3.reshape(1, 1))
    return out


# speedup vs baseline: 1.0384x; 1.0384x over previous
import functools

import numpy as np
import jax
import jax.numpy as jnp
from jax import lax
from jax.experimental import pallas as pl
from jax.experimental.pallas import tpu as pltpu
from jax.experimental.pallas import tpu_sc as plsc

B = 4096
INPUT_ROWS = 260001
F = 26
D = 64
E_COLS = F * D
NPAIR = F * (F - 1) // 2
PAIR_PAD = 384
H1 = 400
H2 = 400
BN_EPS = 1e-3

_OFFSETS = np.arange(F, dtype=np.int32) * 10000

_pairs = [(i, i + o) for o in range(1, F) for i in range(F - o)]
PERM = np.array([i * 25 - i * (i - 1) // 2 + (j - i - 1) for i, j in _pairs],
                dtype=np.int32)

NW = 32
ROWS = B * F
RPW = ROWS // NW
CHUNK = 128
KFIRE = 13
HALF = KFIRE * CHUNK


@functools.lru_cache(maxsize=None)
def _make_gather_sc():
    @functools.partial(
        pl.kernel,
        mesh=plsc.VectorSubcoreMesh(core_axis_name="c", subcore_axis_name="s"),
        out_type=jax.ShapeDtypeStruct((ROWS, D), jnp.float32),
        scratch_types=[
            pltpu.VMEM((RPW,), jnp.int32),
            pltpu.VMEM((HALF, D), jnp.float32),
            pltpu.SemaphoreType.DMA,
        ],
        compiler_params=pltpu.CompilerParams(use_tc_tiling_on_sc=False),
    )
    def _gather_sc(table_hbm, idx_hbm, out_hbm, idx_v, rows_v, sem):
        wid = lax.axis_index("s") * 2 + lax.axis_index("c")
        base = wid * RPW
        pltpu.sync_copy(idx_hbm.at[pl.ds(base, RPW)], idx_v)
        for h in range(RPW // HALF):
            cps = []
            for c in range(KFIRE):
                cp = pltpu.make_async_copy(
                    table_hbm.at[idx_v.at[pl.ds(h * HALF + c * CHUNK, CHUNK)]],
                    rows_v.at[pl.ds(c * CHUNK, CHUNK), :],
                    sem,
                )
                cp.start()
                cps.append(cp)
            for cp in cps:
                cp.wait()
            pltpu.sync_copy(rows_v, out_hbm.at[pl.ds(base + h * HALF, HALF)])

    return _gather_sc


BB = 512

_seg_w = [F - o for o in range(1, F)]
_seg_w8 = [-(-w // 8) * 8 for w in _seg_w]
_SEG_OFF = np.cumsum([0] + _seg_w8[:-1]).astype(np.int32)
PADK = int(np.sum(_seg_w8))
_PAD_POS = np.concatenate(
    [_SEG_OFF[o - 1] + np.arange(F - o) for o in range(1, F)]).astype(np.int32)


def _dot00(a, b):
    return lax.dot_general(a, b, (((0,), (0,)), ((), ())),
                           preferred_element_type=jnp.float32)


def _tc_body(e3_ref, w1a_ref, w1b_ref, s1_ref, b1_ref, w2_ref, s2_ref,
             b2_ref, w3_ref, b3_ref, out_ref):
    et = jnp.concatenate(
        [jnp.transpose(e3_ref[k]) for k in range(F // 2)], axis=0)
    pieces = []
    for o in range(1, F):
        w = F - o
        c = et[:w * D, :] * et[o * D:, :]
        s = jnp.sum(c.reshape(w, D, BB), axis=1)
        pad = _seg_w8[o - 1] - w
        if pad:
            s = jnp.concatenate(
                [s, jnp.zeros((pad, BB), jnp.float32)], axis=0)
        pieces.append(s)
    prod_t = jnp.concatenate(pieces, axis=0)
    h = _dot00(w1a_ref[...], et)
    h += _dot00(w1b_ref[...], prod_t)
    h = jnp.maximum(h * s1_ref[...] + b1_ref[...], 0.0)
    h = _dot00(w2_ref[...], h)
    h = jnp.maximum(h * s2_ref[...] + b2_ref[...], 0.0)
    z = _dot00(w3_ref[...], h) + b3_ref[...]
    out_ref[...] = jnp.transpose(jax.nn.sigmoid(z))


_tc_call = pl.pallas_call(
    _tc_body,
    grid=(B // BB,),
    in_specs=[
        pl.BlockSpec((F // 2, BB, 2 * D), lambda i: (0, i, 0)),
        pl.BlockSpec((E_COLS, H1), lambda i: (0, 0)),
        pl.BlockSpec((PADK, H1), lambda i: (0, 0)),
        pl.BlockSpec((H1, 1), lambda i: (0, 0)),
        pl.BlockSpec((H1, 1), lambda i: (0, 0)),
        pl.BlockSpec((H1, H2), lambda i: (0, 0)),
        pl.BlockSpec((H2, 1), lambda i: (0, 0)),
        pl.BlockSpec((H2, 1), lambda i: (0, 0)),
        pl.BlockSpec((H2, 1), lambda i: (0, 0)),
        pl.BlockSpec((1, 1), lambda i: (0, 0)),
    ],
    out_specs=pl.BlockSpec((BB, 1), lambda i: (i, 0)),
    out_shape=jax.ShapeDtypeStruct((B, 1), jnp.float32),
)


def kernel(x, emb_table, W1, b1, g1, be1, W2, b2, g2, be2, W3, b3):
    m = (x.astype(jnp.int32) + _OFFSETS[None, :]).reshape(B, F // 2, 2)
    idx = jnp.transpose(m, (1, 0, 2)).reshape(-1)
    e = _make_gather_sc()(emb_table, idx)

    s1 = (g1 * (1.0 / np.sqrt(1.0 + BN_EPS))).reshape(H1, 1)
    b1f = b1.reshape(H1, 1) * s1 + be1.reshape(H1, 1)
    s2 = (g2 * (1.0 / np.sqrt(1.0 + BN_EPS))).reshape(H2, 1)
    b2f = b2.reshape(H2, 1) * s2 + be2.reshape(H2, 1)
    w1bp = jnp.zeros((PADK, H1), jnp.float32).at[_PAD_POS].set(
        W1[E_COLS:][PERM])

    out = _tc_call(e.reshape(F // 2, B, 2 * D),
                   W1, w1bp, s1, b1f, W2, s2, b2f, W3, b3.reshape(1, 1))
    return out


# speedup vs baseline: 1.0990x; 1.0584x over previous
import functools

import numpy as np
import jax
import jax.numpy as jnp
from jax import lax
from jax.experimental import pallas as pl
from jax.experimental.pallas import tpu as pltpu
from jax.experimental.pallas import tpu_sc as plsc

B = 4096
INPUT_ROWS = 260001
F = 26
D = 64
E_COLS = F * D
NPAIR = F * (F - 1) // 2
PAIR_PAD = 384
H1 = 400
H2 = 400
BN_EPS = 1e-3

_OFFSETS = np.arange(F, dtype=np.int32) * 10000

_kk = np.repeat(np.arange(F // 2, dtype=np.int32), 2 * B)
_bb = np.tile(np.repeat(np.arange(B, dtype=np.int32), 2), F // 2)
_pp = np.tile(np.array([0, 1], dtype=np.int32), B * (F // 2))
_PERMG = _bb * F + 2 * _kk + _pp

_pairs = [(i, i + o) for o in range(1, F) for i in range(F - o)]
PERM = np.array([i * 25 - i * (i - 1) // 2 + (j - i - 1) for i, j in _pairs],
                dtype=np.int32)

NW = 32
ROWS = B * F
RPW = ROWS // NW
CHUNK = 128
KFIRE = 13
HALF = KFIRE * CHUNK


@functools.lru_cache(maxsize=None)
def _make_gather_sc():
    @functools.partial(
        pl.kernel,
        mesh=plsc.VectorSubcoreMesh(core_axis_name="c", subcore_axis_name="s"),
        out_type=jax.ShapeDtypeStruct((ROWS, D), jnp.float32),
        scratch_types=[
            pltpu.VMEM((RPW,), jnp.int32),
            pltpu.VMEM((HALF, D), jnp.float32),
            pltpu.SemaphoreType.DMA,
        ],
        compiler_params=pltpu.CompilerParams(use_tc_tiling_on_sc=False),
    )
    def _gather_sc(table_hbm, idx_hbm, out_hbm, idx_v, rows_v, sem):
        wid = lax.axis_index("s") * 2 + lax.axis_index("c")
        base = wid * RPW
        pltpu.sync_copy(idx_hbm.at[pl.ds(base, RPW)], idx_v)
        for h in range(RPW // HALF):
            cps = []
            for c in range(KFIRE):
                cp = pltpu.make_async_copy(
                    table_hbm.at[idx_v.at[pl.ds(h * HALF + c * CHUNK, CHUNK)]],
                    rows_v.at[pl.ds(c * CHUNK, CHUNK), :],
                    sem,
                )
                cp.start()
                cps.append(cp)
            for cp in cps:
                cp.wait()
            pltpu.sync_copy(rows_v, out_hbm.at[pl.ds(base + h * HALF, HALF)])

    return _gather_sc


BB = 512

_seg_w = [F - o for o in range(1, F)]
_seg_w8 = [-(-w // 8) * 8 for w in _seg_w]
_SEG_OFF = np.cumsum([0] + _seg_w8[:-1]).astype(np.int32)
PADK = int(np.sum(_seg_w8))
_PAD_POS = np.concatenate(
    [_SEG_OFF[o - 1] + np.arange(F - o) for o in range(1, F)]).astype(np.int32)


def _dot00(a, b):
    return lax.dot_general(a, b, (((0,), (0,)), ((), ())),
                           preferred_element_type=jnp.float32)


def _tc_body(e3_ref, w1a_ref, w1b_ref, s1_ref, b1_ref, w2_ref, s2_ref,
             b2_ref, w3_ref, b3_ref, out_ref):
    et = jnp.concatenate(
        [jnp.transpose(e3_ref[k]) for k in range(F // 2)], axis=0)
    pieces = []
    for o in range(1, F):
        w = F - o
        c = et[:w * D, :] * et[o * D:, :]
        s = jnp.sum(c.reshape(w, D, BB), axis=1)
        pad = _seg_w8[o - 1] - w
        if pad:
            s = jnp.concatenate(
                [s, jnp.zeros((pad, BB), jnp.float32)], axis=0)
        pieces.append(s)
    prod_t = jnp.concatenate(pieces, axis=0)
    h = _dot00(w1a_ref[...], et)
    h += _dot00(w1b_ref[...], prod_t)
    h = jnp.maximum(h * s1_ref[...] + b1_ref[...], 0.0)
    h = _dot00(w2_ref[...], h)
    h = jnp.maximum(h * s2_ref[...] + b2_ref[...], 0.0)
    z = _dot00(w3_ref[...], h) + b3_ref[...]
    out_ref[...] = jnp.transpose(jax.nn.sigmoid(z))


_tc_call = pl.pallas_call(
    _tc_body,
    grid=(B // BB,),
    in_specs=[
        pl.BlockSpec((F // 2, BB, 2 * D), lambda i: (0, i, 0)),
        pl.BlockSpec((E_COLS, H1), lambda i: (0, 0)),
        pl.BlockSpec((PADK, H1), lambda i: (0, 0)),
        pl.BlockSpec((H1, 1), lambda i: (0, 0)),
        pl.BlockSpec((H1, 1), lambda i: (0, 0)),
        pl.BlockSpec((H1, H2), lambda i: (0, 0)),
        pl.BlockSpec((H2, 1), lambda i: (0, 0)),
        pl.BlockSpec((H2, 1), lambda i: (0, 0)),
        pl.BlockSpec((H2, 1), lambda i: (0, 0)),
        pl.BlockSpec((1, 1), lambda i: (0, 0)),
    ],
    out_specs=pl.BlockSpec((BB, 1), lambda i: (i, 0)),
    out_shape=jax.ShapeDtypeStruct((B, 1), jnp.float32),
)


def kernel(x, emb_table, W1, b1, g1, be1, W2, b2, g2, be2, W3, b3):
    m_flat = (x.astype(jnp.int32) + _OFFSETS[None, :]).reshape(-1)
    idx = m_flat[_PERMG]
    e = _make_gather_sc()(emb_table, idx)

    s1 = (g1 * (1.0 / np.sqrt(1.0 + BN_EPS))).reshape(H1, 1)
    b1f = b1.reshape(H1, 1) * s1 + be1.reshape(H1, 1)
    s2 = (g2 * (1.0 / np.sqrt(1.0 + BN_EPS))).reshape(H2, 1)
    b2f = b2.reshape(H2, 1) * s2 + be2.reshape(H2, 1)
    w1bp = jnp.zeros((PADK, H1), jnp.float32).at[_PAD_POS].set(
        W1[E_COLS:][PERM])

    out = _tc_call(e.reshape(F // 2, B, 2 * D),
                   W1, w1bp, s1, b1f, W2, s2, b2f, W3, b3.reshape(1, 1))
    return out


# speedup vs baseline: 1.1025x; 1.0032x over previous
import functools

import numpy as np
import jax
import jax.numpy as jnp
from jax import lax
from jax.experimental import pallas as pl
from jax.experimental.pallas import tpu as pltpu
from jax.experimental.pallas import tpu_sc as plsc

B = 4096
INPUT_ROWS = 260001
F = 26
D = 64
E_COLS = F * D
NPAIR = F * (F - 1) // 2
PAIR_PAD = 384
H1 = 400
H2 = 400
BN_EPS = 1e-3

_OFFSETS = np.arange(F, dtype=np.int32) * 10000

_kk = np.repeat(np.arange(F // 2, dtype=np.int32), 2 * B)
_bb = np.tile(np.repeat(np.arange(B, dtype=np.int32), 2), F // 2)
_pp = np.tile(np.array([0, 1], dtype=np.int32), B * (F // 2))
_PERMG = _bb * F + 2 * _kk + _pp

_pairs = [(i, i + o) for o in range(1, F) for i in range(F - o)]
PERM = np.array([i * 25 - i * (i - 1) // 2 + (j - i - 1) for i, j in _pairs],
                dtype=np.int32)

NW = 32
ROWS = B * F
RPW = ROWS // NW
CHUNK = 128
KFIRE = 13
HALF = KFIRE * CHUNK


@functools.lru_cache(maxsize=None)
def _make_gather_sc():
    @functools.partial(
        pl.kernel,
        mesh=plsc.VectorSubcoreMesh(core_axis_name="c", subcore_axis_name="s"),
        out_type=jax.ShapeDtypeStruct((ROWS, D), jnp.float32),
        scratch_types=[
            pltpu.VMEM((RPW,), jnp.int32),
            pltpu.VMEM((HALF, D), jnp.float32),
            pltpu.SemaphoreType.DMA,
        ],
        compiler_params=pltpu.CompilerParams(use_tc_tiling_on_sc=False),
    )
    def _gather_sc(table_hbm, idx_hbm, out_hbm, idx_v, rows_v, sem):
        wid = lax.axis_index("s") * 2 + lax.axis_index("c")
        base = wid * RPW
        pltpu.sync_copy(idx_hbm.at[pl.ds(base, RPW)], idx_v)
        for h in range(RPW // HALF):
            cps = []
            for c in range(KFIRE):
                cp = pltpu.make_async_copy(
                    table_hbm.at[idx_v.at[pl.ds(h * HALF + c * CHUNK, CHUNK)]],
                    rows_v.at[pl.ds(c * CHUNK, CHUNK), :],
                    sem,
                )
                cp.start()
                cps.append(cp)
            for cp in cps:
                cp.wait()
            pltpu.sync_copy(rows_v, out_hbm.at[pl.ds(base + h * HALF, HALF)])

    return _gather_sc


BB = 1024

_seg_w = [F - o for o in range(1, F)]
_seg_w8 = [-(-w // 8) * 8 for w in _seg_w]
_SEG_OFF = np.cumsum([0] + _seg_w8[:-1]).astype(np.int32)
PADK = int(np.sum(_seg_w8))
_PAD_POS = np.concatenate(
    [_SEG_OFF[o - 1] + np.arange(F - o) for o in range(1, F)]).astype(np.int32)


def _dot00(a, b):
    return lax.dot_general(a, b, (((0,), (0,)), ((), ())),
                           preferred_element_type=jnp.float32)


def _tc_body(e3_ref, w1a_ref, w1b_ref, s1_ref, b1_ref, w2_ref, s2_ref,
             b2_ref, w3_ref, b3_ref, out_ref):
    et = jnp.concatenate(
        [jnp.transpose(e3_ref[k]) for k in range(F // 2)], axis=0)
    pieces = []
    for o in range(1, F):
        w = F - o
        c = et[:w * D, :] * et[o * D:, :]
        s = jnp.sum(c.reshape(w, D, BB), axis=1)
        pad = _seg_w8[o - 1] - w
        if pad:
            s = jnp.concatenate(
                [s, jnp.zeros((pad, BB), jnp.float32)], axis=0)
        pieces.append(s)
    prod_t = jnp.concatenate(pieces, axis=0)
    h = _dot00(w1a_ref[...], et)
    h += _dot00(w1b_ref[...], prod_t)
    h = jnp.maximum(h * s1_ref[...] + b1_ref[...], 0.0)
    h = _dot00(w2_ref[...], h)
    h = jnp.maximum(h * s2_ref[...] + b2_ref[...], 0.0)
    z = _dot00(w3_ref[...], h) + b3_ref[...]
    out_ref[...] = jnp.transpose(jax.nn.sigmoid(z))


_tc_call = pl.pallas_call(
    _tc_body,
    grid=(B // BB,),
    in_specs=[
        pl.BlockSpec((F // 2, BB, 2 * D), lambda i: (0, i, 0)),
        pl.BlockSpec((E_COLS, H1), lambda i: (0, 0)),
        pl.BlockSpec((PADK, H1), lambda i: (0, 0)),
        pl.BlockSpec((H1, 1), lambda i: (0, 0)),
        pl.BlockSpec((H1, 1), lambda i: (0, 0)),
        pl.BlockSpec((H1, H2), lambda i: (0, 0)),
        pl.BlockSpec((H2, 1), lambda i: (0, 0)),
        pl.BlockSpec((H2, 1), lambda i: (0, 0)),
        pl.BlockSpec((H2, 1), lambda i: (0, 0)),
        pl.BlockSpec((1, 1), lambda i: (0, 0)),
    ],
    out_specs=pl.BlockSpec((BB, 1), lambda i: (i, 0)),
    out_shape=jax.ShapeDtypeStruct((B, 1), jnp.float32),
)


def kernel(x, emb_table, W1, b1, g1, be1, W2, b2, g2, be2, W3, b3):
    m_flat = (x.astype(jnp.int32) + _OFFSETS[None, :]).reshape(-1)
    idx = m_flat[_PERMG]
    e = _make_gather_sc()(emb_table, idx)

    s1 = (g1 * (1.0 / np.sqrt(1.0 + BN_EPS))).reshape(H1, 1)
    b1f = b1.reshape(H1, 1) * s1 + be1.reshape(H1, 1)
    s2 = (g2 * (1.0 / np.sqrt(1.0 + BN_EPS))).reshape(H2, 1)
    b2f = b2.reshape(H2, 1) * s2 + be2.reshape(H2, 1)
    w1bp = jnp.zeros((PADK, H1), jnp.float32).at[_PAD_POS].set(
        W1[E_COLS:][PERM])

    out = _tc_call(e.reshape(F // 2, B, 2 * D),
                   W1, w1bp, s1, b1f, W2, s2, b2f, W3, b3.reshape(1, 1))
    return out
